# baseline (speedup 1.0000x reference)
import functools

import jax
import jax.numpy as jnp
from jax import lax
from jax.experimental import pallas as pl
from jax.experimental.pallas import tpu as pltpu
from jax.experimental.pallas import tpu_sc as plsc

_D = 1000
_DP = 1024
_NC, _NS = 2, 16
_NW = _NC * _NS
_BATCH = 1024
_SEQ = 50
_SEQP = 56
_LSTR = 64
_BPW = _BATCH // _NW
_LPW = _BPW * _LSTR
_NCHUNK = _D // 16 + 1


def _gather_rows(table_p, ilist):
  mesh = plsc.VectorSubcoreMesh(core_axis_name="c", subcore_axis_name="s")

  @functools.partial(
      pl.kernel,
      mesh=mesh,
      out_type=jax.ShapeDtypeStruct((_BATCH, _SEQ, _D), jnp.float32),
      scratch_types=[
          pltpu.VMEM((_LPW,), jnp.int32),
          pltpu.VMEM((_SEQP, _DP), jnp.float32),
          pltpu.VMEM((_SEQ, _D), jnp.float32),
          pltpu.SemaphoreType.DMA,
          pltpu.SemaphoreType.DMA,
      ],
  )
  def body(table_hbm, ilist_hbm, out_hbm, ilist_v, pad_v, packed_v, gsem, ssem):
    wid = lax.axis_index("s") * _NC + lax.axis_index("c")
    base = pl.multiple_of(wid * _LPW, 8)
    pltpu.sync_copy(ilist_hbm.at[pl.ds(base, _LPW)], ilist_v)

    def gather_desc(j):
      off = pl.multiple_of(j * _LSTR, 8)
      return pltpu.make_async_copy(
          table_hbm.at[ilist_v.at[pl.ds(off, _SEQP)]], pad_v, gsem
      )

    def scatter_desc(j):
      return pltpu.make_async_copy(
          packed_v, out_hbm.at[wid * _BPW + j], ssem
      )

    def repack():
      @plsc.parallel_loop(0, _SEQ, 1, unroll=4)
      def _row_copy(r):
        for m in range(_NCHUNK):
          off = _D - 16 if m == _NCHUNK - 1 else 16 * m
          packed_v[r, pl.ds(off, 16)] = pad_v[r, pl.ds(off, 16)]

    gather_desc(0).start()

    def batch_step(j, carry):
      gather_desc(j).wait()

      @pl.when(j > 0)
      def _():
        scatter_desc(j - 1).wait()

      repack()

      @pl.when(j + 1 < _BPW)
      def _():
        gather_desc(j + 1).start()

      scatter_desc(j).start()
      return carry

    lax.fori_loop(0, _BPW, batch_step, 0)
    scatter_desc(_BPW - 1).wait()

  return body(table_p, ilist)


def kernel(x, table):
  xi = x.astype(jnp.int32)
  ilist = jnp.concatenate(
      [xi, jnp.broadcast_to(xi[:, _SEQ - 1 :], (_BATCH, _LSTR - _SEQ))], axis=1
  ).reshape(-1)
  table_p = jnp.pad(table, ((0, 0), (0, _DP - _D)))
  return _gather_rows(table_p, ilist)


# speedup vs baseline: 1.4164x; 1.4164x over previous
import functools

import jax
import jax.numpy as jnp
from jax import lax
from jax.experimental import pallas as pl
from jax.experimental.pallas import tpu as pltpu
from jax.experimental.pallas import tpu_sc as plsc

_D = 1000
_DP = 1024
_NC, _NS = 2, 16
_NW = _NC * _NS
_BATCH = 1024
_SEQ = 50
_SEQP = 56
_LSTR = 64
_BPW = _BATCH // _NW
_LPW = _BPW * _LSTR


def _gather_rows(table_p, ilist):
  mesh = plsc.VectorSubcoreMesh(core_axis_name="c", subcore_axis_name="s")

  @functools.partial(
      pl.kernel,
      mesh=mesh,
      out_type=jax.ShapeDtypeStruct((_BATCH, _SEQP, _DP), jnp.float32),
      scratch_types=[
          pltpu.VMEM((_LPW,), jnp.int32),
          pltpu.VMEM((2, _SEQP, _DP), jnp.float32),
          pltpu.SemaphoreType.DMA((2,)),
          pltpu.SemaphoreType.DMA((2,)),
      ],
  )
  def body(table_hbm, ilist_hbm, out_hbm, ilist_v, rows_v, gsem, ssem):
    wid = lax.axis_index("s") * _NC + lax.axis_index("c")
    base = pl.multiple_of(wid * _LPW, 8)
    pltpu.sync_copy(ilist_hbm.at[pl.ds(base, _LPW)], ilist_v)

    def gather_desc(j, b):
      off = pl.multiple_of(j * _LSTR, 8)
      return pltpu.make_async_copy(
          table_hbm.at[ilist_v.at[pl.ds(off, _SEQP)]], rows_v.at[b], gsem.at[b]
      )

    def scatter_desc(j, b):
      return pltpu.make_async_copy(
          rows_v.at[b], out_hbm.at[wid * _BPW + j], ssem.at[b]
      )

    gather_desc(0, 0).start()
    gather_desc(1, 1).start()

    def batch_step(j, carry):
      b = lax.rem(j, 2)
      gather_desc(j, b).wait()
      scatter_desc(j, b).start()

      @pl.when(j + 2 < _BPW)
      def _():
        scatter_desc(j, b).wait()
        gather_desc(j + 2, b).start()

      return carry

    lax.fori_loop(0, _BPW, batch_step, 0)
    scatter_desc(_BPW - 2, 0).wait()
    scatter_desc(_BPW - 1, 1).wait()

  return body(table_p, ilist)


def kernel(x, table):
  xi = x.astype(jnp.int32)
  ilist = jnp.concatenate(
      [xi, jnp.broadcast_to(xi[:, _SEQ - 1 :], (_BATCH, _LSTR - _SEQ))], axis=1
  ).reshape(-1)
  table_p = jnp.pad(table, ((0, 0), (0, _DP - _D)))
  padded = _gather_rows(table_p, ilist)
  return padded[:, :_SEQ, :_D]
